# baseline (speedup 1.0000x reference)
import functools

import jax
import jax.numpy as jnp
from jax import lax
from jax.experimental import pallas as pl
from jax.experimental.pallas import tpu as pltpu
from jax.experimental.pallas import tpu_sc as plsc

N_F = 100000
N_C = 100000
E = 1600000
H = 64
W = 128
NCORE = 2
NSUB = 16
NW = NCORE * NSUB
CHUNK = 11264
NPASS = 9
N_OUT = CHUNK * NPASS
G = 128
STAGE = 2048
NSLOT = STAGE // G
EPW = 51200
NSTAGE = EPW // STAGE
E_PAD = EPW * NW
STRIPE = CHUNK // NSUB
RB = 2000
GRID = N_F // RB


def _dot(a, b):
    return jax.lax.dot(a, b, precision=jax.lax.Precision.HIGHEST)


def _enc_body(ff, cf, few1, feb1, few2, feb2, cew1, ceb1, cew2, ceb2,
              fh_ref, tab_ref):
    fh = _dot(jnp.maximum(_dot(ff[...], few1[...]) + feb1[...], 0.0),
              few2[...]) + feb2[...]
    fh_ref[...] = fh
    ch = _dot(jnp.maximum(_dot(cf[...], cew1[...]) + ceb1[...], 0.0),
              cew2[...]) + ceb2[...]
    pad = jnp.concatenate(
        [jnp.ones((RB, 1), jnp.float32), jnp.zeros((RB, W - H - 1), jnp.float32)],
        axis=1)
    tab_ref[...] = jnp.concatenate([ch, pad], axis=1)


def _encoders(ff, cf, few1, feb1, few2, feb2, cew1, ceb1, cew2, ceb2):
    full = lambda a: pl.BlockSpec(a.shape, lambda i: (i * 0,) * a.ndim)
    return pl.pallas_call(
        _enc_body,
        grid=(GRID,),
        in_specs=[
            pl.BlockSpec((RB, 10), lambda i: (i, i * 0)),
            pl.BlockSpec((RB, 5), lambda i: (i, i * 0)),
            full(few1), full(feb1), full(few2), full(feb2),
            full(cew1), full(ceb1), full(cew2), full(ceb2),
        ],
        out_specs=[
            pl.BlockSpec((RB, H), lambda i: (i, i * 0)),
            pl.BlockSpec((RB, W), lambda i: (i, i * 0)),
        ],
        out_shape=[
            jax.ShapeDtypeStruct((N_F, H), jnp.float32),
            jax.ShapeDtypeStruct((N_C, W), jnp.float32),
        ],
    )(ff, cf, few1, feb1, few2, feb2, cew1, ceb1, cew2, ceb2)


def _segsum_body(tab_hbm, src_hbm, dst_hbm, out_hbm,
                 src_v, dst_v, gdx0, sdx0, rows0, zero_v, acc_sh):
    gdx_v = [gdx0]
    sdx_v = [sdx0]
    rows_v = [rows0]
    c = lax.axis_index("c")
    s = lax.axis_index("s")
    wid = s * NCORE + c
    ebase = wid * EPW
    zero16 = jnp.zeros((16,), jnp.float32)
    zrows = zero_v.shape[0]

    @pl.loop(jnp.int32(0), jnp.int32(zrows))
    def _zero_init(r):
        for j in range(W // 16):
            zero_v[r, pl.ds(j * 16, 16)] = zero16

    @pl.loop(jnp.int32(0), jnp.int32(NPASS))
    def _per_pass(p):
        lo = p * CHUNK
        for z in range(STRIPE // zrows):
            pltpu.sync_copy(zero_v,
                            acc_sh.at[pl.ds(s * STRIPE + z * zrows, zrows)])
        plsc.subcore_barrier()

        @pl.loop(jnp.int32(0), jnp.int32(NSTAGE))
        def _per_stage(t):
            off = ebase + t * STAGE
            pltpu.sync_copy(src_hbm.at[pl.ds(off, STAGE)], src_v)
            pltpu.sync_copy(dst_hbm.at[pl.ds(off, STAGE)], dst_v)

            def _build(b):
                p = 0
                for j in range(G // 16):
                    o = b * G + j * 16
                    sv = src_v[pl.ds(o, 16)]
                    dv = dst_v[pl.ds(o, 16)]
                    rel = sv - lo
                    m = (rel >= 0) & (rel < CHUNK)
                    gdx_v[p][pl.ds(j * 16, 16)] = jnp.where(m, dv, 0)
                    sdx_v[p][pl.ds(j * 16, 16)] = jnp.where(m, rel, CHUNK)

            for b in range(NSLOT):
                _build(b)
                pltpu.sync_copy(tab_hbm.at[gdx_v[0]], rows_v[0])
                pltpu.sync_copy(rows_v[0], acc_sh.at[sdx_v[0]], add=True)

        plsc.subcore_barrier()
        pltpu.sync_copy(acc_sh.at[pl.ds(s * STRIPE, STRIPE)],
                        out_hbm.at[c, pl.ds(lo + s * STRIPE, STRIPE)])
        plsc.subcore_barrier()


def _segsum(tab, src, dst):
    mesh = plsc.VectorSubcoreMesh(core_axis_name="c", subcore_axis_name="s",
                                  num_cores=NCORE, num_subcores=NSUB)
    return pl.kernel(
        _segsum_body,
        out_type=jax.ShapeDtypeStruct((NCORE, N_OUT, W), jnp.float32),
        mesh=mesh,
        scratch_types=[
            pltpu.VMEM((STAGE,), jnp.int32),
            pltpu.VMEM((STAGE,), jnp.int32),
            pltpu.VMEM((G,), jnp.int32),
            pltpu.VMEM((G,), jnp.int32),
            pltpu.VMEM((G, W), jnp.float32),
            pltpu.VMEM((88, W), jnp.float32),
            pltpu.VMEM_SHARED((CHUNK + 16, W), jnp.float32),
        ],
    )(tab, src, dst)


def _tail_body(sums, fh, w1a, w1b, b1, w2, b2, ew1, eb1, ew2, eb2, out_ref):
    st = sums[0] + sums[1]
    cnt = st[:, H:H + 1]
    mean = st[:, :H] / jnp.maximum(cnt, 1.0)
    upd = _dot(fh[...], w1a[...]) + _dot(mean, w1b[...]) + b1[...]
    h = jnp.where(cnt > 0.0, upd, fh[...])
    h = jnp.maximum(_dot(h, w2[...]) + b2[...], 0.0)
    e = _dot(jnp.maximum(_dot(h, ew1[...]) + eb1[...], 0.0), ew2[...]) + eb2[...]
    out_ref[...] = e


def _tail(sums, fh, w1a, w1b, b1, w2, b2, ew1, eb1, ew2, eb2):
    full = lambda a: pl.BlockSpec(a.shape, lambda i: (i * 0,) * a.ndim)
    return pl.pallas_call(
        _tail_body,
        grid=(GRID,),
        in_specs=[
            pl.BlockSpec((NCORE, RB, W), lambda i: (i * 0, i, i * 0)),
            pl.BlockSpec((RB, H), lambda i: (i, i * 0)),
            full(w1a), full(w1b), full(b1), full(w2), full(b2),
            full(ew1), full(eb1), full(ew2), full(eb2),
        ],
        out_specs=pl.BlockSpec((RB, 1), lambda i: (i, i * 0)),
        out_shape=jax.ShapeDtypeStruct((N_F, 1), jnp.float32),
    )(sums, fh, w1a, w1b, b1, w2, b2, ew1, eb1, ew2, eb2)


def kernel(fact_features, constraint_features, fact_constraint_edges,
           fe_w1, fe_b1, fe_w2, fe_b2, ce_w1, ce_b1, ce_w2, ce_b2,
           fc1_w, fc1_b, fc2_w, fc2_b, es_w1, es_b1, es_w2, es_b2):
    src = fact_constraint_edges[0].astype(jnp.int32)
    dst = fact_constraint_edges[1].astype(jnp.int32)
    padlen = E_PAD - E
    src = jnp.concatenate([src, jnp.full((padlen,), -1, jnp.int32)])
    dst = jnp.concatenate([dst, jnp.zeros((padlen,), jnp.int32)])
    f = lambda a: a.astype(jnp.float32)
    (fact_features, constraint_features, fe_w1, fe_b1, fe_w2, fe_b2, ce_w1,
     ce_b1, ce_w2, ce_b2, fc1_w, fc1_b, fc2_w, fc2_b, es_w1, es_b1, es_w2,
     es_b2) = map(f, (fact_features, constraint_features, fe_w1, fe_b1, fe_w2,
                      fe_b2, ce_w1, ce_b1, ce_w2, ce_b2, fc1_w, fc1_b, fc2_w,
                      fc2_b, es_w1, es_b1, es_w2, es_b2))
    r = lambda b: b.reshape(1, -1)
    fh, tab = _encoders(fact_features, constraint_features,
                        fe_w1, r(fe_b1), fe_w2, r(fe_b2),
                        ce_w1, r(ce_b1), ce_w2, r(ce_b2))
    partials = _segsum(tab, src, dst)
    out = _tail(partials, fh, fc1_w[:H], fc1_w[H:], r(fc1_b),
                fc2_w, r(fc2_b), es_w1, r(es_b1), es_w2, r(es_b2))
    return out.reshape(-1).astype(jnp.float64)


# speedup vs baseline: 7.7955x; 7.7955x over previous
import jax
import jax.numpy as jnp
import numpy as np
from jax import lax
from jax.experimental import pallas as pl
from jax.experimental.pallas import tpu as pltpu
from jax.experimental.pallas import tpu_sc as plsc

N_F = 100000
N_C = 100000
E = 1600000
H = 64
WC = 16
NCORE = 2
NSUB = 16
NW = NCORE * NSUB
CHUNK = 20480
NPASS = 5
N_OUT = CHUNK * NPASS
G = 64
STAGE = 2048
NSLOT = STAGE // G
EPW = 51200
NSTAGE = EPW // STAGE
E_PAD = EPW * NW
STRIPE = CHUNK // NSUB
ZR = 160
RB = 2000
GRID = N_F // RB

_PERM = np.zeros(H, np.int32)
for _j in range(H // 32):
    for _i in range(16):
        _PERM[32 * _j + 2 * _i] = 32 * _j + _i
        _PERM[32 * _j + 2 * _i + 1] = 32 * _j + 16 + _i


def _dot(a, b):
    return jax.lax.dot(a, b, precision=jax.lax.Precision.HIGHEST)


def _enc_body(ff, cf, few1, feb1, few2, feb2, cew1, ceb1, cew2, ceb2,
              fh_ref, tab_ref):
    fh = _dot(jnp.maximum(_dot(ff[...], few1[...]) + feb1[...], 0.0),
              few2[...]) + feb2[...]
    fh_ref[...] = fh
    ch = _dot(jnp.maximum(_dot(cf[...], cew1[...]) + ceb1[...], 0.0),
              cew2[...]) + ceb2[...]
    tab_ref[...] = ch.astype(jnp.bfloat16)


def _encoders(ff, cf, few1, feb1, few2, feb2, cew1, ceb1, cew2, ceb2):
    full = lambda a: pl.BlockSpec(a.shape, lambda i: (i * 0,) * a.ndim)
    return pl.pallas_call(
        _enc_body,
        grid=(GRID,),
        in_specs=[
            pl.BlockSpec((RB, 10), lambda i: (i, i * 0)),
            pl.BlockSpec((RB, 5), lambda i: (i, i * 0)),
            full(few1), full(feb1), full(few2), full(feb2),
            full(cew1), full(ceb1), full(cew2), full(ceb2),
        ],
        out_specs=[
            pl.BlockSpec((RB, H), lambda i: (i, i * 0)),
            pl.BlockSpec((RB, H), lambda i: (i, i * 0)),
        ],
        out_shape=[
            jax.ShapeDtypeStruct((N_F, H), jnp.float32),
            jax.ShapeDtypeStruct((N_C, H), jnp.bfloat16),
        ],
    )(ff, cf, few1, feb1, few2, feb2, cew1, ceb1, cew2, ceb2)


def _segsum_body(tab_hbm, src_hbm, dst_hbm, outf_hbm, outc_hbm,
                 src_v, dst_v, gdx_v, sdx_v, rowsb_v, rowsf_v, ones_v,
                 zf_v, zc_v, accf_sh, accc_sh):
    c = lax.axis_index("c")
    s = lax.axis_index("s")
    wid = s * NCORE + c
    ebase = wid * EPW
    zero16 = jnp.zeros((16,), jnp.float32)
    one16 = jnp.ones((16,), jnp.float32)
    mask_hi = jnp.full((16,), -65536, jnp.int32)

    @pl.loop(jnp.int32(0), jnp.int32(ZR))
    def _zero_init(r):
        for j in range(H // 16):
            zf_v[r, pl.ds(j * 16, 16)] = zero16
        zc_v[r, pl.ds(0, WC)] = zero16

    @pl.loop(jnp.int32(0), jnp.int32(G))
    def _ones_init(r):
        ones_v[r, pl.ds(0, WC)] = one16

    @pl.loop(jnp.int32(0), jnp.int32(NPASS))
    def _per_pass(p):
        lo = p * CHUNK
        for z in range(STRIPE // ZR):
            pltpu.sync_copy(zf_v, accf_sh.at[pl.ds(s * STRIPE + z * ZR, ZR)])
            pltpu.sync_copy(zc_v, accc_sh.at[pl.ds(s * STRIPE + z * ZR, ZR)])
        plsc.subcore_barrier()

        @pl.loop(jnp.int32(0), jnp.int32(NSTAGE))
        def _per_stage(t):
            off = ebase + t * STAGE
            pltpu.sync_copy(src_hbm.at[pl.ds(off, STAGE)], src_v)
            pltpu.sync_copy(dst_hbm.at[pl.ds(off, STAGE)], dst_v)
            for b in range(NSLOT):
                for j in range(G // 16):
                    o = b * G + j * 16
                    sv = src_v[pl.ds(o, 16)]
                    dv = dst_v[pl.ds(o, 16)]
                    rel = sv - lo
                    m = (rel >= 0) & (rel < CHUNK)
                    gdx_v[pl.ds(j * 16, 16)] = jnp.where(m, dv, 0)
                    sdx_v[pl.ds(j * 16, 16)] = jnp.where(m, rel, CHUNK)
                pltpu.sync_copy(tab_hbm.at[gdx_v], rowsb_v)

                @pl.loop(jnp.int32(0), jnp.int32(G))
                def _unpack(r):
                    for j in range(H // 32):
                        v = rowsb_v[r, pl.ds(j * 16, 16)]
                        lo32 = jax.lax.bitcast_convert_type(v << 16,
                                                            jnp.float32)
                        hi32 = jax.lax.bitcast_convert_type(v & mask_hi,
                                                            jnp.float32)
                        rowsf_v[r, pl.ds(j * 32, 16)] = lo32
                        rowsf_v[r, pl.ds(j * 32 + 16, 16)] = hi32

                pltpu.sync_copy(rowsf_v, accf_sh.at[sdx_v], add=True)
                pltpu.sync_copy(ones_v, accc_sh.at[sdx_v], add=True)

        plsc.subcore_barrier()
        pltpu.sync_copy(accf_sh.at[pl.ds(s * STRIPE, STRIPE)],
                        outf_hbm.at[c, pl.ds(lo + s * STRIPE, STRIPE)])
        pltpu.sync_copy(accc_sh.at[pl.ds(s * STRIPE, STRIPE)],
                        outc_hbm.at[c, pl.ds(lo + s * STRIPE, STRIPE)])
        plsc.subcore_barrier()


def _segsum(tab, src, dst):
    mesh = plsc.VectorSubcoreMesh(core_axis_name="c", subcore_axis_name="s",
                                  num_cores=NCORE, num_subcores=NSUB)
    return pl.kernel(
        _segsum_body,
        out_type=[
            jax.ShapeDtypeStruct((NCORE, N_OUT, H), jnp.float32),
            jax.ShapeDtypeStruct((NCORE, N_OUT, WC), jnp.float32),
        ],
        mesh=mesh,
        compiler_params=pltpu.CompilerParams(use_tc_tiling_on_sc=False),
        scratch_types=[
            pltpu.VMEM((STAGE,), jnp.int32),
            pltpu.VMEM((STAGE,), jnp.int32),
            pltpu.VMEM((G,), jnp.int32),
            pltpu.VMEM((G,), jnp.int32),
            pltpu.VMEM((G, H // 2), jnp.int32),
            pltpu.VMEM((G, H), jnp.float32),
            pltpu.VMEM((G, WC), jnp.float32),
            pltpu.VMEM((ZR, H), jnp.float32),
            pltpu.VMEM((ZR, WC), jnp.float32),
            pltpu.VMEM_SHARED((CHUNK + 16, H), jnp.float32),
            pltpu.VMEM_SHARED((CHUNK + 16, WC), jnp.float32),
        ],
    )(tab, src, dst)


def _tail_body(sums, cnts, fh, w1a, w1b, b1, w2, b2, ew1, eb1, ew2, eb2,
               out_ref):
    st = sums[0] + sums[1]
    cnt = cnts[0][:, :1] + cnts[1][:, :1]
    mean = st / jnp.maximum(cnt, 1.0)
    upd = _dot(fh[...], w1a[...]) + _dot(mean, w1b[...]) + b1[...]
    h = jnp.where(cnt > 0.0, upd, fh[...])
    h = jnp.maximum(_dot(h, w2[...]) + b2[...], 0.0)
    e = _dot(jnp.maximum(_dot(h, ew1[...]) + eb1[...], 0.0), ew2[...]) + eb2[...]
    out_ref[...] = e


def _tail(sums, cnts, fh, w1a, w1b, b1, w2, b2, ew1, eb1, ew2, eb2):
    full = lambda a: pl.BlockSpec(a.shape, lambda i: (i * 0,) * a.ndim)
    return pl.pallas_call(
        _tail_body,
        grid=(GRID,),
        in_specs=[
            pl.BlockSpec((NCORE, RB, H), lambda i: (i * 0, i, i * 0)),
            pl.BlockSpec((NCORE, RB, WC), lambda i: (i * 0, i, i * 0)),
            pl.BlockSpec((RB, H), lambda i: (i, i * 0)),
            full(w1a), full(w1b), full(b1), full(w2), full(b2),
            full(ew1), full(eb1), full(ew2), full(eb2),
        ],
        out_specs=pl.BlockSpec((RB, 1), lambda i: (i, i * 0)),
        out_shape=jax.ShapeDtypeStruct((N_F, 1), jnp.float32),
    )(sums, cnts, fh, w1a, w1b, b1, w2, b2, ew1, eb1, ew2, eb2)


def kernel(fact_features, constraint_features, fact_constraint_edges,
           fe_w1, fe_b1, fe_w2, fe_b2, ce_w1, ce_b1, ce_w2, ce_b2,
           fc1_w, fc1_b, fc2_w, fc2_b, es_w1, es_b1, es_w2, es_b2):
    src = fact_constraint_edges[0].astype(jnp.int32)
    dst = fact_constraint_edges[1].astype(jnp.int32)
    padlen = E_PAD - E
    src = jnp.concatenate([src, jnp.full((padlen,), -1, jnp.int32)])
    dst = jnp.concatenate([dst, jnp.zeros((padlen,), jnp.int32)])
    f = lambda a: a.astype(jnp.float32)
    (fact_features, constraint_features, fe_w1, fe_b1, fe_w2, fe_b2, ce_w1,
     ce_b1, ce_w2, ce_b2, fc1_w, fc1_b, fc2_w, fc2_b, es_w1, es_b1, es_w2,
     es_b2) = map(f, (fact_features, constraint_features, fe_w1, fe_b1, fe_w2,
                      fe_b2, ce_w1, ce_b1, ce_w2, ce_b2, fc1_w, fc1_b, fc2_w,
                      fc2_b, es_w1, es_b1, es_w2, es_b2))
    r = lambda b: b.reshape(1, -1)
    perm = jnp.asarray(_PERM)
    ce_w2p = ce_w2[:, perm]
    ce_b2p = ce_b2[perm]
    fh, tab = _encoders(fact_features, constraint_features,
                        fe_w1, r(fe_b1), fe_w2, r(fe_b2),
                        ce_w1, r(ce_b1), ce_w2p, r(ce_b2p))
    tab_i32 = jax.lax.bitcast_convert_type(
        tab.reshape(N_C, H // 2, 2), jnp.int32)
    partials, counts = _segsum(tab_i32, src, dst)
    out = _tail(partials, counts, fh, fc1_w[:H], fc1_w[H:], r(fc1_b),
                fc2_w, r(fc2_b), es_w1, r(es_b1), es_w2, r(es_b2))
    return out.reshape(-1).astype(jnp.float64)


# speedup vs baseline: 135.7639x; 17.4158x over previous
import jax
import jax.numpy as jnp
import numpy as np
from jax import lax
from jax.experimental import pallas as pl
from jax.experimental.pallas import tpu as pltpu
from jax.experimental.pallas import tpu_sc as plsc
from jax._src.config import enable_x64 as _enable_x64

N_F = 100000
N_C = 100000
E = 1600000
H = 64
WC = 16
NCORE = 2
NSUB = 16
NW = NCORE * NSUB
CHUNK = 20480
NPASS = 5
N_OUT = CHUNK * NPASS
G = 64
STAGE = 2048
NSLOT = STAGE // G
EPW = 51200
NSTAGE = EPW // STAGE
E_PAD = EPW * NW
STRIPE = CHUNK // NSUB
ZR = 32
RB = 2000
GRID = N_F // RB

_PERM = np.zeros(H, np.int32)
for _j in range(H // 32):
    for _i in range(16):
        _PERM[32 * _j + 2 * _i] = 32 * _j + _i
        _PERM[32 * _j + 2 * _i + 1] = 32 * _j + 16 + _i


def _dot(a, b):
    return jax.lax.dot(a, b, precision=jax.lax.Precision.HIGHEST)


def _enc_body(ff, cf, few1, feb1, few2, feb2, cew1, ceb1, cew2, ceb2,
              fh_ref, tab_ref):
    fh = _dot(jnp.maximum(_dot(ff[...], few1[...]) + feb1[...], 0.0),
              few2[...]) + feb2[...]
    fh_ref[...] = fh
    ch = _dot(jnp.maximum(_dot(cf[...], cew1[...]) + ceb1[...], 0.0),
              cew2[...]) + ceb2[...]
    tab_ref[...] = ch.astype(jnp.bfloat16)


def _encoders(ff, cf, few1, feb1, few2, feb2, cew1, ceb1, cew2, ceb2):
    full = lambda a: pl.BlockSpec(a.shape, lambda i: (i * 0,) * a.ndim)
    return pl.pallas_call(
        _enc_body,
        grid=(GRID,),
        in_specs=[
            pl.BlockSpec((RB, 10), lambda i: (i, i * 0)),
            pl.BlockSpec((RB, 5), lambda i: (i, i * 0)),
            full(few1), full(feb1), full(few2), full(feb2),
            full(cew1), full(ceb1), full(cew2), full(ceb2),
        ],
        out_specs=[
            pl.BlockSpec((RB, H), lambda i: (i, i * 0)),
            pl.BlockSpec((RB, H), lambda i: (i, i * 0)),
        ],
        out_shape=[
            jax.ShapeDtypeStruct((N_F, H), jnp.float32),
            jax.ShapeDtypeStruct((N_C, H), jnp.bfloat16),
        ],
    )(ff, cf, few1, feb1, few2, feb2, cew1, ceb1, cew2, ceb2)


def _dg(x, p):
    return lax.gather(
        x, p[:, None],
        dimension_numbers=lax.GatherDimensionNumbers(
            offset_dims=(), collapsed_slice_dims=(0,), start_index_map=(0,)),
        slice_sizes=(1,), mode=lax.GatherScatterMode.PROMISE_IN_BOUNDS)


def _segsum_body(tab_hbm, src_hbm, dst_hbm, outf_hbm, outc_hbm,
                 src_v, dst_v, cd_v, cl_v, gdx_v, sdx_v, rowsb_v, rowsf_v,
                 ones_v, zf_v, zc_v, accf_sh, accc_sh):
    c = lax.axis_index("c")
    s = lax.axis_index("s")
    wid = s * NCORE + c
    ebase = wid * EPW
    zero16 = jnp.zeros((16,), jnp.float32)
    one16 = jnp.ones((16,), jnp.float32)
    mask_hi = jnp.full((16,), -65536, jnp.int32)

    @pl.loop(jnp.int32(0), jnp.int32(ZR))
    def _zero_init(r):
        for j in range(H // 16):
            zf_v[r, pl.ds(j * 16, 16)] = zero16
        zc_v[r, pl.ds(0, WC)] = zero16

    @pl.loop(jnp.int32(0), jnp.int32(G))
    def _ones_init(r):
        ones_v[r, pl.ds(0, WC)] = one16

    @pl.loop(jnp.int32(0), jnp.int32(NPASS))
    def _per_pass(p):
        lo = p * CHUNK
        for z in range(STRIPE // ZR):
            pltpu.sync_copy(zf_v, accf_sh.at[pl.ds(s * STRIPE + z * ZR, ZR)])
            pltpu.sync_copy(zc_v, accc_sh.at[pl.ds(s * STRIPE + z * ZR, ZR)])
        plsc.subcore_barrier()

        @pl.loop(jnp.int32(0), jnp.int32(NSTAGE))
        def _per_stage(t):
            off = ebase + t * STAGE
            pltpu.sync_copy(src_hbm.at[pl.ds(off, STAGE)], src_v)
            pltpu.sync_copy(dst_hbm.at[pl.ds(off, STAGE)], dst_v)

            pay = jnp.int32(0x1FFFF)
            lane = lax.iota(jnp.int32, 16)

            def _scan(i, cnt):
                sv = src_v[pl.ds(i * 16, 16)]
                dv = dst_v[pl.ds(i * 16, 16)]
                rel = sv - lo
                m = (rel >= 0) & (rel < CHUNK)
                flag = jnp.where(m, jnp.int32(0), jnp.int32(1)) << 21
                lb = lane << 17
                w1 = flag | lb | jnp.where(m, dv, 0)
                w2 = flag | lb | jnp.where(m, rel, 0)
                for k in (2, 4, 8, 16):
                    j = k >> 1
                    while j >= 1:
                        p = lane ^ j
                        pw1 = _dg(w1, p)
                        pw2 = _dg(w2, p)
                        up_i = jnp.where((lane & k) == 0, 1, 0)
                        low_i = jnp.where(lane < p, 1, 0)
                        lt_i = jnp.where(pw1 < w1, 1, 0)
                        gt_i = jnp.where(pw1 > w1, 1, 0)
                        c_i = jnp.where(up_i == low_i, lt_i, gt_i)
                        take = c_i == 1
                        w1 = jnp.where(take, pw1, w1)
                        w2 = jnp.where(take, pw2, w2)
                        j >>= 1
                mi = jnp.where(m, jnp.int32(1), jnp.int32(0))
                for j in (1, 2, 4, 8):
                    mi = mi + _dg(mi, lane ^ j)
                cd_v[pl.ds(cnt, 16)] = w1 & pay
                cl_v[pl.ds(cnt, 16)] = w2 & pay
                return cnt + mi[0]

            n = lax.fori_loop(jnp.int32(0), jnp.int32(STAGE // 16), _scan,
                              jnp.int32(0))
            dummy_d = jnp.zeros((16,), jnp.int32)
            dummy_l = jnp.full((16,), CHUNK, jnp.int32)
            for j in range(G // 16):
                cd_v[pl.ds(n + j * 16, 16)] = dummy_d
                cl_v[pl.ds(n + j * 16, 16)] = dummy_l
            for b in range(NSLOT):
                slot_live = jnp.int32(b * G) < n

                @pl.when(slot_live)
                def _slot():
                    for j in range(G // 16):
                        gdx_v[pl.ds(j * 16, 16)] = cd_v[pl.ds(b * G + j * 16, 16)]
                        sdx_v[pl.ds(j * 16, 16)] = cl_v[pl.ds(b * G + j * 16, 16)]
                    pltpu.sync_copy(tab_hbm.at[gdx_v], rowsb_v)

                    @pl.loop(jnp.int32(0), jnp.int32(G))
                    def _unpack(r):
                        for j in range(H // 32):
                            v = rowsb_v[r, pl.ds(j * 16, 16)]
                            lo32 = jax.lax.bitcast_convert_type(v << 16,
                                                                jnp.float32)
                            hi32 = jax.lax.bitcast_convert_type(v & mask_hi,
                                                                jnp.float32)
                            rowsf_v[r, pl.ds(j * 32, 16)] = lo32
                            rowsf_v[r, pl.ds(j * 32 + 16, 16)] = hi32

                    pltpu.sync_copy(rowsf_v, accf_sh.at[sdx_v], add=True)
                    pltpu.sync_copy(ones_v, accc_sh.at[sdx_v], add=True)

        plsc.subcore_barrier()
        pltpu.sync_copy(accf_sh.at[pl.ds(s * STRIPE, STRIPE)],
                        outf_hbm.at[c, pl.ds(lo + s * STRIPE, STRIPE)])
        pltpu.sync_copy(accc_sh.at[pl.ds(s * STRIPE, STRIPE)],
                        outc_hbm.at[c, pl.ds(lo + s * STRIPE, STRIPE)])
        plsc.subcore_barrier()


def _segsum(tab, src, dst):
    mesh = plsc.VectorSubcoreMesh(core_axis_name="c", subcore_axis_name="s",
                                  num_cores=NCORE, num_subcores=NSUB)
    return pl.kernel(
        _segsum_body,
        out_type=[
            jax.ShapeDtypeStruct((NCORE, N_OUT, H), jnp.float32),
            jax.ShapeDtypeStruct((NCORE, N_OUT, WC), jnp.float32),
        ],
        mesh=mesh,
        compiler_params=pltpu.CompilerParams(use_tc_tiling_on_sc=False),
        scratch_types=[
            pltpu.VMEM((STAGE,), jnp.int32),
            pltpu.VMEM((STAGE,), jnp.int32),
            pltpu.VMEM((STAGE + 2 * G,), jnp.int32),
            pltpu.VMEM((STAGE + 2 * G,), jnp.int32),
            pltpu.VMEM((G,), jnp.int32),
            pltpu.VMEM((G,), jnp.int32),
            pltpu.VMEM((G, H // 2), jnp.int32),
            pltpu.VMEM((G, H), jnp.float32),
            pltpu.VMEM((G, WC), jnp.float32),
            pltpu.VMEM((ZR, H), jnp.float32),
            pltpu.VMEM((ZR, WC), jnp.float32),
            pltpu.VMEM_SHARED((CHUNK + 16, H), jnp.float32),
            pltpu.VMEM_SHARED((CHUNK + 16, WC), jnp.float32),
        ],
    )(tab, src, dst)


def _tail_body(sums, cnts, fh, w1a, w1b, b1, w2, b2, ew1, eb1, ew2, eb2,
               out_ref):
    st = sums[0] + sums[1]
    cnt = cnts[0][:, :1] + cnts[1][:, :1]
    mean = st / jnp.maximum(cnt, 1.0)
    upd = _dot(fh[...], w1a[...]) + _dot(mean, w1b[...]) + b1[...]
    h = jnp.where(cnt > 0.0, upd, fh[...])
    h = jnp.maximum(_dot(h, w2[...]) + b2[...], 0.0)
    e = _dot(jnp.maximum(_dot(h, ew1[...]) + eb1[...], 0.0), ew2[...]) + eb2[...]
    out_ref[...] = e


def _tail(sums, cnts, fh, w1a, w1b, b1, w2, b2, ew1, eb1, ew2, eb2):
    full = lambda a: pl.BlockSpec(a.shape, lambda i: (i * 0,) * a.ndim)
    return pl.pallas_call(
        _tail_body,
        grid=(GRID,),
        in_specs=[
            pl.BlockSpec((NCORE, RB, H), lambda i: (i * 0, i, i * 0)),
            pl.BlockSpec((NCORE, RB, WC), lambda i: (i * 0, i, i * 0)),
            pl.BlockSpec((RB, H), lambda i: (i, i * 0)),
            full(w1a), full(w1b), full(b1), full(w2), full(b2),
            full(ew1), full(eb1), full(ew2), full(eb2),
        ],
        out_specs=pl.BlockSpec((RB, 1), lambda i: (i, i * 0)),
        out_shape=jax.ShapeDtypeStruct((N_F, 1), jnp.float32),
    )(sums, cnts, fh, w1a, w1b, b1, w2, b2, ew1, eb1, ew2, eb2)


def kernel(fact_features, constraint_features, fact_constraint_edges,
           fe_w1, fe_b1, fe_w2, fe_b2, ce_w1, ce_b1, ce_w2, ce_b2,
           fc1_w, fc1_b, fc2_w, fc2_b, es_w1, es_b1, es_w2, es_b2):
    with _enable_x64(False):
        out32 = _impl(fact_features, constraint_features,
                      fact_constraint_edges, fe_w1, fe_b1, fe_w2, fe_b2,
                      ce_w1, ce_b1, ce_w2, ce_b2, fc1_w, fc1_b, fc2_w,
                      fc2_b, es_w1, es_b1, es_w2, es_b2)
    return out32.astype(jnp.float64)


def _impl(fact_features, constraint_features, fact_constraint_edges,
          fe_w1, fe_b1, fe_w2, fe_b2, ce_w1, ce_b1, ce_w2, ce_b2,
          fc1_w, fc1_b, fc2_w, fc2_b, es_w1, es_b1, es_w2, es_b2):
    src = fact_constraint_edges[0].astype(jnp.int32)
    dst = fact_constraint_edges[1].astype(jnp.int32)
    padlen = E_PAD - E
    src = jnp.concatenate([src, jnp.full((padlen,), -1, jnp.int32)])
    dst = jnp.concatenate([dst, jnp.zeros((padlen,), jnp.int32)])
    f = lambda a: a.astype(jnp.float32)
    (fact_features, constraint_features, fe_w1, fe_b1, fe_w2, fe_b2, ce_w1,
     ce_b1, ce_w2, ce_b2, fc1_w, fc1_b, fc2_w, fc2_b, es_w1, es_b1, es_w2,
     es_b2) = map(f, (fact_features, constraint_features, fe_w1, fe_b1, fe_w2,
                      fe_b2, ce_w1, ce_b1, ce_w2, ce_b2, fc1_w, fc1_b, fc2_w,
                      fc2_b, es_w1, es_b1, es_w2, es_b2))
    r = lambda b: b.reshape(1, -1)
    perm = jnp.asarray(_PERM)
    ce_w2p = ce_w2[:, perm]
    ce_b2p = ce_b2[perm]
    fh, tab = _encoders(fact_features, constraint_features,
                        fe_w1, r(fe_b1), fe_w2, r(fe_b2),
                        ce_w1, r(ce_b1), ce_w2p, r(ce_b2p))
    tab_i32 = jax.lax.bitcast_convert_type(
        tab.reshape(N_C, H // 2, 2), jnp.int32)
    partials, counts = _segsum(tab_i32, src, dst)
    out = _tail(partials, counts, fh, fc1_w[:H], fc1_w[H:], r(fc1_b),
                fc2_w, r(fc2_b), es_w1, r(es_b1), es_w2, r(es_b2))
    return out.reshape(-1)
